# baseline (speedup 1.0000x reference)
import jax
import jax.numpy as jnp
from jax import lax
from jax.experimental import pallas as pl
from jax.experimental.pallas import tpu as pltpu
from jax.experimental.pallas import tpu_sc as plsc

DIM = 128
NC, NS, L = 2, 16, 16
NW = NC * NS
E = 320000
PER_W = E // NW
GR = 80
NG = PER_W // GR
NB = 5
LAG = 3
HEAD = 2 * NB * GR
S0, S1 = 21, 3


def _body(a0_hbm, a1_hbm, a2_hbm, tab_hbm, out_hbm,
          a0v, a1v, a2v, cidx, rb0, rb1, rb2, rb3, rb4, tab_sp,
          isem, rsem, g0, g1, g2, g3, g4, s0, s1, s2, s3, s4):
    wid = lax.axis_index("s") * NC + lax.axis_index("c")
    base = pl.multiple_of(wid * PER_W, GR)
    rows = [rb0, rb1, rb2, rb3, rb4]
    gsem = [g0, g1, g2, g3, g4]
    ssem = [s0, s1, s2, s3, s4]

    d0 = pltpu.async_copy(a0_hbm.at[pl.ds(base, HEAD)], a0v.at[pl.ds(0, HEAD)], isem)
    d1 = pltpu.async_copy(a1_hbm.at[pl.ds(base, HEAD)], a1v.at[pl.ds(0, HEAD)], isem)
    d2 = pltpu.async_copy(a2_hbm.at[pl.ds(base, HEAD)], a2v.at[pl.ds(0, HEAD)], isem)
    base_r = pl.multiple_of(base + HEAD, 8)
    r0 = pltpu.async_copy(a0_hbm.at[pl.ds(base_r, PER_W - HEAD)],
                          a0v.at[pl.ds(HEAD, PER_W - HEAD)], rsem)
    r1 = pltpu.async_copy(a1_hbm.at[pl.ds(base_r, PER_W - HEAD)],
                          a1v.at[pl.ds(HEAD, PER_W - HEAD)], rsem)
    r2 = pltpu.async_copy(a2_hbm.at[pl.ds(base_r, PER_W - HEAD)],
                          a2v.at[pl.ds(HEAD, PER_W - HEAD)], rsem)
    sid = lax.axis_index("s")
    srow = pl.multiple_of(sid * (128 // NS), 8)
    pltpu.sync_copy(tab_hbm.at[pl.ds(srow, 128 // NS)],
                    tab_sp.at[pl.ds(srow, 128 // NS)])
    d0.wait(); d1.wait(); d2.wait()
    plsc.subcore_barrier()

    def cidx_slice(t):
        return cidx.at[pl.ds(pl.multiple_of(t * GR, 8), GR)]

    def prep(t, b):
        for j in range(GR // L):
            s = pl.ds(pl.multiple_of(t * GR + j * L, 8), L)
            cidx[s] = a0v[s] * S0 + a1v[s] * S1 + a2v[s]

    def fire_gather(t, b):
        pltpu.async_copy(tab_sp.at[cidx_slice(t)], rows[b], gsem[b])

    def finish(t, b):
        pltpu.make_async_copy(tab_sp.at[cidx_slice(t)], rows[b], gsem[b]).wait()
        off = pl.multiple_of(base + t * GR, 8)
        pltpu.async_copy(rows[b], out_hbm.at[pl.ds(off, GR)], ssem[b])

    def drain_store(b):
        pltpu.make_async_copy(rows[b], out_hbm.at[pl.ds(base, GR)], ssem[b]).wait()

    for t in range(NB):
        prep(t, t)
        fire_gather(t, t)
        if t >= LAG:
            finish(t - LAG, t - LAG)
    r0.wait(); r1.wait(); r2.wait()

    def outer(i, carry):
        t0 = NB + i * NB
        for u in range(NB):
            t = t0 + u
            b = u
            prep(t, b)
            finish(t - LAG, (u - LAG) % NB)
            drain_store(b)
            fire_gather(t, b)
        return carry

    lax.fori_loop(0, NG // NB - 1, outer, 0)

    for t in range(NG - LAG, NG):
        finish(t, t % NB)
    for b in range(NB):
        drain_store(b)


def kernel(edge_attr, W0, W1, W2):
    ea = edge_attr.astype(jnp.int32)
    a0, a1, a2 = ea[:, 0], ea[:, 1], ea[:, 2]
    tab = (W0[:, None, None, :] + W1[None, :, None, :]
           + W2[None, None, :, :]).reshape(-1, DIM)
    tab = jnp.pad(tab, ((0, 128 - tab.shape[0]), (0, 0)))

    mesh = plsc.VectorSubcoreMesh(core_axis_name="c", subcore_axis_name="s")
    f = pl.kernel(
        _body,
        out_type=jax.ShapeDtypeStruct((E, DIM), jnp.float32),
        mesh=mesh,
        scratch_types=(
            [pltpu.VMEM((PER_W,), jnp.int32) for _ in range(4)]
            + [pltpu.VMEM((GR, DIM), jnp.float32) for _ in range(NB)]
            + [pltpu.VMEM_SHARED((128, DIM), jnp.float32)]
            + [pltpu.SemaphoreType.DMA for _ in range(12)]
        ),
    )
    return f(a0, a1, a2, tab)
